# baseline (speedup 1.0000x reference)
import functools

import jax
import jax.numpy as jnp
from jax import lax
from jax.experimental import pallas as pl
from jax.experimental.pallas import tpu as pltpu
from jax.experimental.pallas import tpu_sc as plsc

E = 64
K = 8
H = 4096
T = 8192
T_BLK = 512
N_WORKERS = 32

_SORT8 = [(0, 1), (2, 3), (4, 5), (6, 7),
          (0, 2), (1, 3), (4, 6), (5, 7), (1, 2), (5, 6),
          (0, 4), (1, 5), (2, 6), (3, 7), (2, 4), (3, 5),
          (1, 2), (3, 4), (5, 6)]
_BITONIC8 = [(0, 4), (1, 5), (2, 6), (3, 7),
             (0, 2), (1, 3), (4, 6), (5, 7),
             (0, 1), (2, 3), (4, 5), (6, 7)]


def _tc_body(x_ref, w_ref, m_ref, logits_ref, maskedT_ref):
    x = x_ref[...]
    w = w_ref[...]
    lt = lax.dot_general(x, w, (((1,), (1,)), ((), ())),
                         preferred_element_type=jnp.float32)
    logits_ref[...] = lt
    maskedT_ref[...] = (lt * m_ref[...]).T


def _tc_router_chunk(x, w, mask_row, chunk, t_chunk):
    blocks = t_chunk // T_BLK
    block0 = chunk * blocks
    return pl.pallas_call(
        _tc_body,
        grid=(blocks,),
        in_specs=[
            pl.BlockSpec((T_BLK, H), lambda i: (block0 + i, 0)),
            pl.BlockSpec((E, H), lambda i: (0, 0)),
            pl.BlockSpec((1, E), lambda i: (0, 0)),
        ],
        out_specs=[
            pl.BlockSpec((T_BLK, E), lambda i: (i, 0)),
            pl.BlockSpec((E, T_BLK), lambda i: (0, i)),
        ],
        out_shape=[
            jax.ShapeDtypeStruct((t_chunk, E), jnp.float32),
            jax.ShapeDtypeStruct((E, t_chunk), jnp.float32),
        ],
    )(x, w, mask_row)


def _ce(p, q):
    m = p[0] >= q[0]
    hi = (jnp.where(m, p[0], q[0]), jnp.where(m, p[1], q[1]))
    lo = (jnp.where(m, q[0], p[0]), jnp.where(m, q[1], p[1]))
    return hi, lo


def _sort8(a):
    a = list(a)
    for i, j in _SORT8:
        a[i], a[j] = _ce(a[i], a[j])
    return a


def _merge_top8(a, b):
    w = []
    for i in range(8):
        m = a[i][0] >= b[7 - i][0]
        w.append((jnp.where(m, a[i][0], b[7 - i][0]),
                  jnp.where(m, a[i][1], b[7 - i][1])))
    for i, j in _BITONIC8:
        w[i], w[j] = _ce(w[i], w[j])
    return w


@functools.cache
def _make_sc_topk(t_chunk):
    rows_per_w = t_chunk // N_WORKERS
    groups = rows_per_w // 16

    @functools.partial(
        pl.kernel,
        mesh=plsc.VectorSubcoreMesh(core_axis_name="c", subcore_axis_name="s"),
        out_type=[
            jax.ShapeDtypeStruct((N_WORKERS, K, rows_per_w), jnp.float32),
            jax.ShapeDtypeStruct((N_WORKERS, K, rows_per_w), jnp.int32),
        ],
        scratch_types=[
            pltpu.VMEM((E, rows_per_w), jnp.float32),
            pltpu.VMEM((K, rows_per_w), jnp.float32),
            pltpu.VMEM((K, rows_per_w), jnp.int32),
        ],
    )
    def _sc_topk(maskedT_hbm, rw_hbm, se_hbm, ltb, wv, iv):
        wid = lax.axis_index("s") * 2 + lax.axis_index("c")
        base = wid * rows_per_w
        pltpu.sync_copy(maskedT_hbm.at[:, pl.ds(base, rows_per_w)], ltb)

        def group(g, carry):
            col0 = g * 16

            def load_block(b):
                blk = []
                for t in range(8):
                    e = b * 8 + t
                    v = ltb[e, pl.ds(col0, 16)]
                    i = jnp.full((16,), e, jnp.int32)
                    blk.append((v, i))
                return blk

            top = _sort8(load_block(0))
            for b in range(1, 8):
                top = _merge_top8(top, _sort8(load_block(b)))

            s = top[0][0]
            for j in range(1, K):
                s = s + top[j][0]
            inv = 1.0 / s
            for j in range(K):
                wv[j, pl.ds(col0, 16)] = top[j][0] * inv
                iv[j, pl.ds(col0, 16)] = top[j][1]
            return carry

        lax.fori_loop(0, groups, group, 0)
        pltpu.sync_copy(wv, rw_hbm.at[wid])
        pltpu.sync_copy(iv, se_hbm.at[wid])

    return _sc_topk


def kernel(hidden_states, W, available_experts):
    mask_row = (
        jnp.zeros((E,), jnp.float32).at[available_experts].set(1.0).reshape(1, E)
    )
    router_logits, maskedT = _tc_router_chunk(hidden_states, W, mask_row, 0, T)
    rw_kt, se_kt = _make_sc_topk(T)(maskedT)
    routing_weights = rw_kt.transpose(0, 2, 1).reshape(T, K)
    selected_experts = se_kt.transpose(0, 2, 1).reshape(T, K)
    return (router_logits, routing_weights, selected_experts)
